# baseline (speedup 1.0000x reference)
import functools

import jax
import jax.numpy as jnp
from jax import lax
from jax.experimental import pallas as pl
from jax.experimental.pallas import tpu as pltpu
from jax.experimental.pallas import tpu_sc as plsc

VOCAB = 1000
BATCH = 1024
SEQ = 50
SEQPAD = 56
D = VOCAB
DPAD = 1024
NUM_CORES = 2
NUM_SUBCORES = 16
NW = NUM_CORES * NUM_SUBCORES
BPW = BATCH // NW
NSLICE = D // 16
TAILOFF = D - 16


def _make_sc_gather():
    mesh = plsc.VectorSubcoreMesh(core_axis_name="c", subcore_axis_name="s")

    @functools.partial(
        pl.kernel,
        mesh=mesh,
        out_type=jax.ShapeDtypeStruct((BATCH, SEQ, D), jnp.float32),
        scratch_types=[
            pltpu.VMEM((BPW * SEQPAD,), jnp.int32),
            pltpu.VMEM((SEQPAD, DPAD), jnp.float32),
            pltpu.VMEM((SEQ, D), jnp.float32),
            pltpu.SemaphoreType.DMA,
        ],
    )
    def k(table_hbm, idx_hbm, out_hbm, idx_v, gbuf, wbuf, gsem):
        cid = lax.axis_index("c")
        sid = lax.axis_index("s")
        wid = sid * NUM_CORES + cid
        pltpu.sync_copy(idx_hbm.at[pl.ds(wid * BPW * SEQPAD, BPW * SEQPAD)],
                        idx_v)

        def repack_row(r, carry):
            for kk in range(NSLICE):
                wbuf[r, pl.ds(kk * 16, 16)] = gbuf[r, pl.ds(kk * 16, 16)]
            wbuf[r, pl.ds(TAILOFF, 16)] = gbuf[r, pl.ds(TAILOFF, 16)]
            return carry

        def body(j, carry):
            bi = wid * BPW + j
            pltpu.async_copy(
                table_hbm.at[idx_v.at[pl.ds(j * SEQPAD, SEQPAD)]],
                gbuf,
                gsem,
            ).wait()
            lax.fori_loop(0, SEQ, repack_row, 0)
            pltpu.sync_copy(wbuf, out_hbm.at[bi])
            return carry

        lax.fori_loop(0, BPW, body, 0)

    return k


_sc_gather = _make_sc_gather()


def kernel(x, table):
    xpad = jnp.pad(x.astype(jnp.int32), ((0, 0), (0, SEQPAD - SEQ)))
    table_pad = jnp.pad(table, ((0, 0), (0, DPAD - D)))
    return _sc_gather(table_pad, xpad.reshape(-1))
